# baseline (speedup 1.0000x reference)
import functools

import jax
import jax.numpy as jnp
from jax import lax
from jax.experimental import pallas as pl
from jax.experimental.pallas import tpu as pltpu
from jax.experimental.pallas import tpu_sc as plsc

EMBED_DIM = 1024
ADDED_OFFSET = 50000
SINCOS = 1000
VQ_START = 56000
VQ_END = 64192
VQ_VOCAB = 8192
VQ_DIM = 256

NC, NS, L = 2, 16, 16
NW = NC * NS
NTOK = 4 * 4096
BPW = NTOK // NW
K = 32
NCH = BPW // K
FLAT = (NCH + 1) * K
NBUF = 3

RANGES = (
    (0, ADDED_OFFSET),
    (ADDED_OFFSET, ADDED_OFFSET + SINCOS),
    (ADDED_OFFSET + SINCOS, VQ_START),
    (VQ_START, VQ_END),
)


def _sc_scratch(nranges):
    return (
        [pltpu.VMEM((BPW,), jnp.int32)]
        + [pltpu.VMEM((FLAT,), jnp.int32) for _ in range(nranges)]
        + [pltpu.VMEM((FLAT,), jnp.int32) for _ in range(nranges)]
        + [pltpu.VMEM((NCH + 1, K), jnp.int32) for _ in range(nranges)]
        + [pltpu.VMEM((K, EMBED_DIM), jnp.float32) for _ in range(NBUF)]
        + [pltpu.SemaphoreType.DMA for _ in range(2 * NBUF)]
    )


def _sc_common(range_ids, tables, x_hbm, out_hbm,
               ids_v, idxs, poss, pos2d, rows, gsems, ssems):
    nr = len(range_ids)
    wid = lax.axis_index("s") * NC + lax.axis_index("c")
    base = wid * BPW
    pltpu.sync_copy(x_hbm.at[pl.ds(base, BPW)], ids_v)

    lane = lax.iota(jnp.int32, L)

    def compact_step(j, cnts):
        v = ids_v[pl.ds(j * L, L)]
        p = base + j * L + lane
        new_cnts = []
        for k in range(nr):
            lo, hi = RANGES[range_ids[k]]
            m = (v >= lo) & (v < hi)
            mi = m.astype(jnp.int32)
            scan = plsc.cumsum(mi)
            dest = cnts[k] + scan - mi
            plsc.store_scatter(idxs[k], [dest], v - lo, mask=m)
            plsc.store_scatter(poss[k], [dest], p, mask=m)
            new_cnts.append(cnts[k] + scan[L - 1])
        return tuple(new_cnts)

    cnts = lax.fori_loop(
        0, BPW // L, compact_step, (jnp.int32(0),) * nr)

    for k in range(nr):
        @pl.when(cnts[k] > 0)
        def _(k=k):
            di = jnp.broadcast_to(idxs[k][pl.ds(0, L)][0], (L,))
            dp = jnp.broadcast_to(poss[k][pl.ds(0, L)][0], (L,))
            idxs[k][pl.ds(cnts[k], L)] = di
            idxs[k][pl.ds(cnts[k] + L, L)] = di
            poss[k][pl.ds(cnts[k], L)] = dp
            poss[k][pl.ds(cnts[k] + L, L)] = dp

    def repack_step(j, carry):
        for k in range(nr):
            pos2d[k][j, pl.ds(0, L)] = poss[k][pl.ds(j * K, L)]
            pos2d[k][j, pl.ds(L, L)] = poss[k][pl.ds(j * K + L, L)]
        return carry

    lax.fori_loop(0, NCH + 1, repack_step, 0)

    def wait_sem(sem, buf):
        pltpu.make_async_copy(out_hbm.at[pl.ds(0, K)], buf, sem).wait()

    for k in range(nr):
        nch = (cnts[k] + (K - 1)) // K
        ngrp = (nch + (NBUF - 1)) // NBUF

        def grp_step(i, carry, k=k, nch=nch):
            c0 = NBUF * i
            for s in range(NBUF):
                @pl.when(c0 + s < nch)
                def _(s=s):
                    @pl.when(i > 0)
                    def _():
                        wait_sem(ssems[s], rows[s])

                    pltpu.async_copy(
                        tables[k].at[idxs[k].at[pl.ds((c0 + s) * K, K)]],
                        rows[s], gsems[s])

            for s in range(NBUF):
                @pl.when(c0 + s < nch)
                def _(s=s):
                    wait_sem(gsems[s], rows[s])
                    pltpu.async_copy(
                        rows[s], out_hbm.at[pos2d[k].at[c0 + s]], ssems[s])

            return carry

        lax.fori_loop(0, ngrp, grp_step, 0)

        for s in range(NBUF):
            @pl.when(nch > s)
            def _(s=s):
                wait_sem(ssems[s], rows[s])


_SC_MESH = plsc.VectorSubcoreMesh(core_axis_name="c", subcore_axis_name="s")
_SC_PARAMS = pltpu.CompilerParams(needs_layout_passes=False)


@functools.partial(
    pl.kernel,
    out_type=jax.ShapeDtypeStruct((NTOK, EMBED_DIM), jnp.float32),
    mesh=_SC_MESH,
    scratch_types=_sc_scratch(3),
    compiler_params=_SC_PARAMS,
)
def _sc_text(tok_hbm, num_hbm, add_hbm, x_hbm, out_hbm,
             ids_v, i0, i1, i2, p0, p1, p2, q0, q1, q2,
             rows_a, rows_b, rows_c, gsem_a, gsem_b, gsem_c,
             ssem_a, ssem_b, ssem_c):
    _sc_common((0, 1, 2), (tok_hbm, num_hbm, add_hbm), x_hbm, out_hbm,
               ids_v, (i0, i1, i2), (p0, p1, p2), (q0, q1, q2),
               (rows_a, rows_b, rows_c), (gsem_a, gsem_b, gsem_c),
               (ssem_a, ssem_b, ssem_c))


@functools.partial(
    pl.kernel,
    out_type=(),
    mesh=_SC_MESH,
    scratch_types=_sc_scratch(1),
    compiler_params=_SC_PARAMS,
)
def _sc_img(img_hbm, x_hbm, out_hbm,
            ids_v, i0, p0, q0,
            rows_a, rows_b, rows_c, gsem_a, gsem_b, gsem_c,
            ssem_a, ssem_b, ssem_c):
    _sc_common((3,), (img_hbm,), x_hbm, out_hbm,
               ids_v, (i0,), (p0,), (q0,),
               (rows_a, rows_b, rows_c), (gsem_a, gsem_b, gsem_c),
               (ssem_a, ssem_b, ssem_c))


def _proj_body(cb_ref, w_ref, out_ref):
    out_ref[:] = lax.dot_general(
        cb_ref[:], w_ref[:], (((1,), (1,)), ((), ())),
        preferred_element_type=jnp.float32)


def _project(cb, w):
    return pl.pallas_call(
        _proj_body,
        grid=(8,),
        in_specs=[
            pl.BlockSpec((VQ_VOCAB // 8, VQ_DIM), lambda i: (i, 0)),
            pl.BlockSpec((EMBED_DIM, VQ_DIM), lambda i: (0, 0)),
        ],
        out_specs=pl.BlockSpec((VQ_VOCAB // 8, EMBED_DIM), lambda i: (i, 0)),
        out_shape=jax.ShapeDtypeStruct((VQ_VOCAB, EMBED_DIM), jnp.float32),
    )(cb, w)


def kernel(x, token_table, added_table, numbers_table, vqgan_codebook, proj_W):
    img_table = _project(vqgan_codebook, proj_W)
    xf = x.reshape(-1).astype(jnp.int32)
    out = _sc_text(token_table, numbers_table, added_table, xf)
    out_ref = jax.new_ref(out)
    _sc_img(img_table, xf, out_ref)
    return out_ref[...].reshape(x.shape[0], x.shape[1], EMBED_DIM)


# speedup vs baseline: 1.1801x; 1.1801x over previous
import functools

import jax
import jax.numpy as jnp
from jax import lax
from jax.experimental import pallas as pl
from jax.experimental.pallas import tpu as pltpu
from jax.experimental.pallas import tpu_sc as plsc

EMBED_DIM = 1024
ADDED_OFFSET = 50000
SINCOS = 1000
VQ_START = 56000
VQ_END = 64192
VQ_VOCAB = 8192
VQ_DIM = 256

NC, NS, L = 2, 16, 16
NW = NC * NS
NTOK = 4 * 4096
BPW = NTOK // NW
K = 16
NBUF = 6
FLAT = -(-(BPW + 2 * L) // K) * K
NROW2D = FLAT // K

RANGES = (
    (0, ADDED_OFFSET),
    (ADDED_OFFSET, ADDED_OFFSET + SINCOS),
    (ADDED_OFFSET + SINCOS, VQ_START),
    (VQ_START, VQ_END),
)


def _sc_scratch(nranges):
    return (
        [pltpu.VMEM((BPW,), jnp.int32)]
        + [pltpu.VMEM((FLAT,), jnp.int32) for _ in range(nranges)]
        + [pltpu.VMEM((FLAT,), jnp.int32) for _ in range(nranges)]
        + [pltpu.VMEM((NROW2D, K), jnp.int32) for _ in range(nranges)]
        + [pltpu.VMEM((K, EMBED_DIM), jnp.float32) for _ in range(NBUF)]
        + [pltpu.SemaphoreType.DMA for _ in range(2 * NBUF)]
    )


def _sc_common(range_ids, tables, x_hbm, out_hbm,
               ids_v, idxs, poss, pos2d, rows, gsems, ssems):
    nr = len(range_ids)
    wid = lax.axis_index("s") * NC + lax.axis_index("c")
    base = wid * BPW
    pltpu.sync_copy(x_hbm.at[pl.ds(base, BPW)], ids_v)

    lane = lax.iota(jnp.int32, L)

    def compact_step(j, cnts):
        v = ids_v[pl.ds(j * L, L)]
        p = base + j * L + lane
        new_cnts = []
        for k in range(nr):
            lo, hi = RANGES[range_ids[k]]
            m = (v >= lo) & (v < hi)
            mi = m.astype(jnp.int32)
            scan = plsc.cumsum(mi)
            dest = cnts[k] + scan - mi
            plsc.store_scatter(idxs[k], [dest], v - lo, mask=m)
            plsc.store_scatter(poss[k], [dest], p, mask=m)
            new_cnts.append(cnts[k] + scan[L - 1])
        return tuple(new_cnts)

    cnts = lax.fori_loop(
        0, BPW // L, compact_step, (jnp.int32(0),) * nr)

    for k in range(nr):
        @pl.when(cnts[k] > 0)
        def _(k=k):
            di = jnp.broadcast_to(idxs[k][pl.ds(0, L)][0], (L,))
            dp = jnp.broadcast_to(poss[k][pl.ds(0, L)][0], (L,))
            idxs[k][pl.ds(cnts[k], L)] = di
            idxs[k][pl.ds(cnts[k] + L, L)] = di
            poss[k][pl.ds(cnts[k], L)] = dp
            poss[k][pl.ds(cnts[k] + L, L)] = dp

    def repack_step(j, carry):
        for k in range(nr):
            for h in range(K // L):
                pos2d[k][j, pl.ds(h * L, L)] = poss[k][pl.ds(j * K + h * L, L)]
        return carry

    lax.fori_loop(0, NROW2D, repack_step, 0)

    def wait_sem(sem, buf):
        pltpu.make_async_copy(out_hbm.at[pl.ds(0, K)], buf, sem).wait()

    for k in range(nr):
        nch = (cnts[k] + (K - 1)) // K
        ngrp = (nch + (NBUF - 1)) // NBUF

        def grp_step(i, carry, k=k, nch=nch):
            c0 = NBUF * i
            for s in range(NBUF):
                @pl.when(c0 + s < nch)
                def _(s=s):
                    @pl.when(i > 0)
                    def _():
                        wait_sem(ssems[s], rows[s])

                    pltpu.async_copy(
                        tables[k].at[idxs[k].at[pl.ds((c0 + s) * K, K)]],
                        rows[s], gsems[s])

            for s in range(NBUF):
                @pl.when(c0 + s < nch)
                def _(s=s):
                    wait_sem(gsems[s], rows[s])
                    pltpu.async_copy(
                        rows[s], out_hbm.at[pos2d[k].at[c0 + s]], ssems[s])

            return carry

        lax.fori_loop(0, ngrp, grp_step, 0)

        for s in range(NBUF):
            @pl.when(nch > s)
            def _(s=s):
                wait_sem(ssems[s], rows[s])


_SC_MESH = plsc.VectorSubcoreMesh(core_axis_name="c", subcore_axis_name="s")
_SC_PARAMS = pltpu.CompilerParams(needs_layout_passes=False)


def _unpack_scratch(nr, scr):
    ids_v = scr[0]
    idxs = scr[1:1 + nr]
    poss = scr[1 + nr:1 + 2 * nr]
    pos2d = scr[1 + 2 * nr:1 + 3 * nr]
    rows = scr[1 + 3 * nr:1 + 3 * nr + NBUF]
    gsems = scr[1 + 3 * nr + NBUF:1 + 3 * nr + 2 * NBUF]
    ssems = scr[1 + 3 * nr + 2 * NBUF:1 + 3 * nr + 3 * NBUF]
    return ids_v, idxs, poss, pos2d, rows, gsems, ssems


@functools.partial(
    pl.kernel,
    out_type=jax.ShapeDtypeStruct((NTOK, EMBED_DIM), jnp.float32),
    mesh=_SC_MESH,
    scratch_types=_sc_scratch(3),
    compiler_params=_SC_PARAMS,
)
def _sc_text(tok_hbm, num_hbm, add_hbm, x_hbm, out_hbm, *scr):
    _sc_common((0, 1, 2), (tok_hbm, num_hbm, add_hbm), x_hbm, out_hbm,
               *_unpack_scratch(3, scr))


@functools.partial(
    pl.kernel,
    out_type=(),
    mesh=_SC_MESH,
    scratch_types=_sc_scratch(1),
    compiler_params=_SC_PARAMS,
)
def _sc_img(img_hbm, x_hbm, out_hbm, *scr):
    _sc_common((3,), (img_hbm,), x_hbm, out_hbm, *_unpack_scratch(1, scr))


def _proj_body(cb_ref, w_ref, out_ref):
    out_ref[:] = lax.dot_general(
        cb_ref[:], w_ref[:], (((1,), (1,)), ((), ())),
        preferred_element_type=jnp.float32)


def _project(cb, w):
    return pl.pallas_call(
        _proj_body,
        grid=(8,),
        in_specs=[
            pl.BlockSpec((VQ_VOCAB // 8, VQ_DIM), lambda i: (i, 0)),
            pl.BlockSpec((EMBED_DIM, VQ_DIM), lambda i: (0, 0)),
        ],
        out_specs=pl.BlockSpec((VQ_VOCAB // 8, EMBED_DIM), lambda i: (i, 0)),
        out_shape=jax.ShapeDtypeStruct((VQ_VOCAB, EMBED_DIM), jnp.float32),
    )(cb, w)


def kernel(x, token_table, added_table, numbers_table, vqgan_codebook, proj_W):
    img_table = _project(vqgan_codebook, proj_W)
    xf = x.reshape(-1).astype(jnp.int32)
    out = _sc_text(token_table, numbers_table, added_table, xf)
    out_ref = jax.new_ref(out)
    _sc_img(img_table, xf, out_ref)
    return out_ref[...].reshape(x.shape[0], x.shape[1], EMBED_DIM)
